# Initial kernel scaffold; baseline (speedup 1.0000x reference)
import jax
import jax.numpy as jnp
from jax.experimental import pallas as pl


def kernel(u_values, v_values):
    raise NotImplementedError("write your pallas kernel here")


# speedup vs baseline: 54.9354x; 54.9354x over previous
import jax
import jax.numpy as jnp
from jax import lax
from jax.experimental import pallas as pl
from jax.experimental.pallas import tpu as pltpu
from jax.experimental.pallas import tpu_sc as plsc

N = 8192
NV = N // 16
NLEV = 9
ROWS_PER_W = 4
NW = 32
_UNROLL = 4


def _body(u_hbm, v_hbm, out_hbm, buf, res_v):
  c = lax.axis_index("c")
  s = lax.axis_index("s")
  wid = c * 16 + s
  lanes = lax.iota(jnp.int32, 16)

  def vreg(i):
    return pl.ds(i * 16, 16)

  def row_body(r, res):
    row = wid * ROWS_PER_W + r
    pltpu.sync_copy(u_hbm.at[row], buf.at[pl.ds(0, N)])
    pltpu.sync_copy(v_hbm.at[row], buf.at[pl.ds(N, N)])

    @plsc.parallel_loop(0, 2 * NV, unroll=_UNROLL)
    def _init(i):
      buf[vreg(i)] = jnp.sort(buf[vreg(i)])

    for j in range(1, NLEV + 1):
      K = 1 << (j - 1)

      @plsc.parallel_loop(0, NV, unroll=_UNROLL)
      def _stage_a(p):
        arr = p >> 8
        q = p & (NV // 2 - 1)
        m = q >> (j - 1)
        i = q & (K - 1)
        base = arr * NV + m * 2 * K
        vi = base + i
        vj = base + 2 * K - 1 - i
        x = buf[vreg(vi)]
        ry = jnp.flip(buf[vreg(vj)])
        buf[vreg(vi)] = jnp.minimum(x, ry)
        buf[vreg(vj)] = jnp.flip(jnp.maximum(x, ry))

      d = K // 2
      while d >= 1:
        dd = d

        @plsc.parallel_loop(0, NV, unroll=_UNROLL)
        def _stage_b(p):
          arr = p >> 8
          q = p & (NV // 2 - 1)
          blk = q // dd
          off = q & (dd - 1)
          vi = arr * NV + blk * 2 * dd + off
          vj = vi + dd
          x = buf[vreg(vi)]
          y = buf[vreg(vj)]
          buf[vreg(vi)] = jnp.minimum(x, y)
          buf[vreg(vj)] = jnp.maximum(x, y)

        d //= 2

      @plsc.parallel_loop(0, 2 * NV, unroll=_UNROLL)
      def _stage_c(i):
        buf[vreg(i)] = jnp.sort(buf[vreg(i)])

    def acc_body(i, acc):
      return acc + jnp.abs(buf[vreg(i)] - buf[vreg(NV + i)])

    acc = lax.fori_loop(0, NV, acc_body, jnp.zeros((16,), jnp.float32))
    total = jnp.sum(acc) * (1.0 / N)
    return jnp.where(lanes == r, total, res)

  res = lax.fori_loop(0, ROWS_PER_W, row_body, jnp.zeros((16,), jnp.float32))
  res_v[...] = res
  pltpu.sync_copy(res_v, out_hbm.at[wid])


def kernel(u_values, v_values):
  mesh = plsc.VectorSubcoreMesh(core_axis_name="c", subcore_axis_name="s")
  out = pl.kernel(
      _body,
      out_type=jax.ShapeDtypeStruct((NW, 16), jnp.float32),
      mesh=mesh,
      compiler_params=pltpu.CompilerParams(needs_layout_passes=False),
      scratch_types=[
          pltpu.VMEM((2 * N,), jnp.float32),
          pltpu.VMEM((16,), jnp.float32),
      ],
  )(u_values, v_values)
  return out[:, :ROWS_PER_W].reshape(128)


# speedup vs baseline: 72.0956x; 1.3124x over previous
import jax
import jax.numpy as jnp
from jax import lax
from jax.experimental import pallas as pl
from jax.experimental.pallas import tpu as pltpu
from jax.experimental.pallas import tpu_sc as plsc

N = 8192
NV = N // 16
NLEV = 9
ROWS_PER_W = 4
NW = 32
_UNROLL = 4


def _body(u_hbm, v_hbm, out_hbm, buf, res_v):
  c = lax.axis_index("c")
  s = lax.axis_index("s")
  wid = c * 16 + s
  lanes = lax.iota(jnp.int32, 16)

  def vreg(i):
    return pl.ds(i * 16, 16)

  def row_body(r, res):
    row = wid * ROWS_PER_W + r
    pltpu.sync_copy(u_hbm.at[row], buf.at[pl.ds(0, N)])
    pltpu.sync_copy(v_hbm.at[row], buf.at[pl.ds(N, N)])

    @plsc.parallel_loop(0, NV, unroll=_UNROLL)
    def _level1(p):
      vi = 2 * p
      x = jnp.sort(buf[vreg(vi)])
      ry = jnp.flip(jnp.sort(buf[vreg(vi + 1)]))
      buf[vreg(vi)] = jnp.sort(jnp.minimum(x, ry))
      buf[vreg(vi + 1)] = jnp.sort(jnp.maximum(x, ry))

    for j in range(2, NLEV + 1):
      K = 1 << (j - 1)

      @plsc.parallel_loop(0, NV, unroll=_UNROLL)
      def _stage_a(p):
        arr = p >> 8
        q = p & (NV // 2 - 1)
        m = q >> (j - 1)
        i = q & (K - 1)
        base = arr * NV + m * 2 * K
        vi = base + i
        vj = base + 2 * K - 1 - i
        x = buf[vreg(vi)]
        ry = jnp.flip(buf[vreg(vj)])
        buf[vreg(vi)] = jnp.minimum(x, ry)
        buf[vreg(vj)] = jnp.flip(jnp.maximum(x, ry))

      d = K // 2
      while d >= 2:
        dd = d

        @plsc.parallel_loop(0, NV, unroll=_UNROLL)
        def _stage_b(p):
          arr = p >> 8
          q = p & (NV // 2 - 1)
          blk = q // dd
          off = q & (dd - 1)
          vi = arr * NV + blk * 2 * dd + off
          vj = vi + dd
          x = buf[vreg(vi)]
          y = buf[vreg(vj)]
          buf[vreg(vi)] = jnp.minimum(x, y)
          buf[vreg(vj)] = jnp.maximum(x, y)

        d //= 2

      if j < NLEV:

        @plsc.parallel_loop(0, NV, unroll=_UNROLL)
        def _stage_b1(p):
          vi = 2 * p
          x = buf[vreg(vi)]
          y = buf[vreg(vi + 1)]
          buf[vreg(vi)] = jnp.sort(jnp.minimum(x, y))
          buf[vreg(vi + 1)] = jnp.sort(jnp.maximum(x, y))

    @plsc.parallel_loop(
        0, NV // 2, unroll=_UNROLL, carry=jnp.zeros((16,), jnp.float32))
    def _finish(q, acc):
      ui = 2 * q
      xu = buf[vreg(ui)]
      yu = buf[vreg(ui + 1)]
      su0 = jnp.sort(jnp.minimum(xu, yu))
      su1 = jnp.sort(jnp.maximum(xu, yu))
      xv = buf[vreg(NV + ui)]
      yv = buf[vreg(NV + ui + 1)]
      sv0 = jnp.sort(jnp.minimum(xv, yv))
      sv1 = jnp.sort(jnp.maximum(xv, yv))
      return acc + jnp.abs(su0 - sv0) + jnp.abs(su1 - sv1)

    total = jnp.sum(_finish) * (1.0 / N)
    return jnp.where(lanes == r, total, res)

  res = lax.fori_loop(0, ROWS_PER_W, row_body, jnp.zeros((16,), jnp.float32))
  res_v[...] = res
  pltpu.sync_copy(res_v, out_hbm.at[wid])


def kernel(u_values, v_values):
  mesh = plsc.VectorSubcoreMesh(core_axis_name="c", subcore_axis_name="s")
  out = pl.kernel(
      _body,
      out_type=jax.ShapeDtypeStruct((NW, 16), jnp.float32),
      mesh=mesh,
      compiler_params=pltpu.CompilerParams(needs_layout_passes=False),
      scratch_types=[
          pltpu.VMEM((2 * N,), jnp.float32),
          pltpu.VMEM((16,), jnp.float32),
      ],
  )(u_values, v_values)
  return out[:, :ROWS_PER_W].reshape(128)


# speedup vs baseline: 101.8905x; 1.4133x over previous
import jax
import jax.numpy as jnp
from jax import lax
from jax.experimental import pallas as pl
from jax.experimental.pallas import tpu as pltpu
from jax.experimental.pallas import tpu_sc as plsc

N = 8192
NV = N // 16
NLEV = 9
ROWS_PER_W = 4
NW = 32
_UNROLL = 4


def _body(u_hbm, v_hbm, out_hbm, buf, res_v):
  c = lax.axis_index("c")
  s = lax.axis_index("s")
  wid = c * 16 + s
  lanes = lax.iota(jnp.int32, 16)

  def vreg(i):
    return pl.ds(i * 16, 16)

  def row_body(r, res):
    row = wid * ROWS_PER_W + r
    pltpu.sync_copy(u_hbm.at[row], buf.at[pl.ds(0, N)])
    pltpu.sync_copy(v_hbm.at[row], buf.at[pl.ds(N, N)])

    @plsc.parallel_loop(0, NV, unroll=_UNROLL)
    def _level1(p):
      vi = 2 * p
      x = jnp.sort(buf[vreg(vi)])
      ry = jnp.flip(jnp.sort(buf[vreg(vi + 1)]))
      buf[vreg(vi)] = jnp.sort(jnp.minimum(x, ry))
      buf[vreg(vi + 1)] = jnp.sort(jnp.maximum(x, ry))

    def network(xs, dists, final_vsort):
      L = len(dists)
      xs = list(xs)
      for si in range(L):
        st = 1 << (L - 1 - si)
        for t in range(1 << L):
          if (t // st) % 2 == 0:
            a, b = xs[t], xs[t + st]
            xs[t] = jnp.minimum(a, b)
            xs[t + st] = jnp.maximum(a, b)
      if final_vsort:
        xs = [jnp.sort(x) for x in xs]
      return xs

    acc_final = [None]
    for j in range(2, NLEV + 1):
      K = 1 << (j - 1)

      @plsc.parallel_loop(0, NV, unroll=_UNROLL)
      def _stage_a(p):
        arr = p >> 8
        q = p & (NV // 2 - 1)
        m = q >> (j - 1)
        i = q & (K - 1)
        base = arr * NV + m * 2 * K
        vi = base + i
        vj = base + 2 * K - 1 - i
        x = buf[vreg(vi)]
        ry = jnp.flip(buf[vreg(vj)])
        buf[vreg(vi)] = jnp.minimum(x, ry)
        buf[vreg(vj)] = jnp.flip(jnp.maximum(x, ry))

      dists_all = []
      d = K // 2
      while d >= 1:
        dists_all.append(d)
        d //= 2
      chunks = []
      while dists_all:
        take = dists_all[-3:] if len(dists_all) >= 3 else dists_all[:]
        chunks.append(take)
        dists_all = dists_all[: len(dists_all) - len(take)]
      chunks.reverse()

      for ci, dists in enumerate(chunks):
        last = ci == len(chunks) - 1
        L = len(dists)
        G = 1 << L
        b = dists[-1].bit_length() - 1
        acc_mode = last and j == NLEV
        unroll = 2 if G >= 8 else _UNROLL

        if acc_mode:

          @plsc.parallel_loop(
              0, NV // G, unroll=unroll,
              carry=jnp.zeros((16,), jnp.float32))
          def _chunk_acc(g, acc):
            v0 = ((g >> b) << (b + L)) | (g & ((1 << b) - 1))
            xu = network(
                [buf[vreg(v0 + (t << b))] for t in range(G)], dists, True)
            xv = network(
                [buf[vreg(NV + v0 + (t << b))] for t in range(G)], dists,
                True)
            for a_u, a_v in zip(xu, xv):
              acc = acc + jnp.abs(a_u - a_v)
            return acc

          acc_final[0] = _chunk_acc
        else:

          @plsc.parallel_loop(0, 2 * NV // G, unroll=unroll)
          def _chunk(g):
            v0 = ((g >> b) << (b + L)) | (g & ((1 << b) - 1))
            xs = network(
                [buf[vreg(v0 + (t << b))] for t in range(G)], dists, last)
            for t in range(G):
              buf[vreg(v0 + (t << b))] = xs[t]

    total = jnp.sum(acc_final[0]) * (1.0 / N)
    return jnp.where(lanes == r, total, res)

  res = lax.fori_loop(0, ROWS_PER_W, row_body, jnp.zeros((16,), jnp.float32))
  res_v[...] = res
  pltpu.sync_copy(res_v, out_hbm.at[wid])


def kernel(u_values, v_values):
  mesh = plsc.VectorSubcoreMesh(core_axis_name="c", subcore_axis_name="s")
  out = pl.kernel(
      _body,
      out_type=jax.ShapeDtypeStruct((NW, 16), jnp.float32),
      mesh=mesh,
      compiler_params=pltpu.CompilerParams(needs_layout_passes=False),
      scratch_types=[
          pltpu.VMEM((2 * N,), jnp.float32),
          pltpu.VMEM((16,), jnp.float32),
      ],
  )(u_values, v_values)
  return out[:, :ROWS_PER_W].reshape(128)


# speedup vs baseline: 123.1264x; 1.2084x over previous
import jax
import jax.numpy as jnp
from jax import lax
from jax.experimental import pallas as pl
from jax.experimental.pallas import tpu as pltpu
from jax.experimental.pallas import tpu_sc as plsc

N = 8192
NV = N // 16
NLEV = 9
ROWS_PER_W = 4
NW = 32
_UNROLL = 4


def _body(u_hbm, v_hbm, out_hbm, buf, res_v):
  c = lax.axis_index("c")
  s = lax.axis_index("s")
  wid = c * 16 + s
  lanes = lax.iota(jnp.int32, 16)

  def vreg(i):
    return pl.ds(i * 16, 16)

  def row_body(r, res):
    row = wid * ROWS_PER_W + r
    pltpu.sync_copy(u_hbm.at[row], buf.at[pl.ds(0, N)])
    pltpu.sync_copy(v_hbm.at[row], buf.at[pl.ds(N, N)])

    @plsc.parallel_loop(0, NV, unroll=_UNROLL)
    def _level1(p):
      vi = 2 * p
      x = jnp.sort(buf[vreg(vi)])
      ry = jnp.flip(jnp.sort(buf[vreg(vi + 1)]))
      buf[vreg(vi)] = jnp.sort(jnp.minimum(x, ry))
      buf[vreg(vi + 1)] = jnp.sort(jnp.maximum(x, ry))

    def network(xs, dists, final_vsort):
      L = len(dists)
      xs = list(xs)
      for si in range(L):
        st = 1 << (L - 1 - si)
        for t in range(1 << L):
          if (t // st) % 2 == 0:
            a, b = xs[t], xs[t + st]
            xs[t] = jnp.minimum(a, b)
            xs[t + st] = jnp.maximum(a, b)
      if final_vsort:
        xs = [jnp.sort(x) for x in xs]
      return xs

    acc_final = [None]
    for j in range(2, NLEV + 1):
      K = 1 << (j - 1)

      if j == 2:
        @plsc.parallel_loop(0, 2 * NV // 4, unroll=_UNROLL)
        def _level2(g):
          base = 4 * g
          x0 = buf[vreg(base)]
          x1 = buf[vreg(base + 1)]
          r2 = jnp.flip(buf[vreg(base + 2)])
          r3 = jnp.flip(buf[vreg(base + 3)])
          l0 = jnp.minimum(x0, r3)
          h0 = jnp.maximum(x0, r3)
          l1 = jnp.minimum(x1, r2)
          h1 = jnp.maximum(x1, r2)
          buf[vreg(base)] = jnp.sort(jnp.minimum(l0, l1))
          buf[vreg(base + 1)] = jnp.sort(jnp.maximum(l0, l1))
          buf[vreg(base + 2)] = jnp.sort(jnp.minimum(h1, h0))
          buf[vreg(base + 3)] = jnp.sort(jnp.maximum(h1, h0))

        continue

      Q = K // 4

      @plsc.parallel_loop(0, NV // 4, unroll=2)
      def _stage_a(g):
        m = g // Q
        i = g & (Q - 1)
        base = m * 2 * K
        lower = [base + i + t * Q for t in range(4)]
        upper = [base + 2 * K - 1 - i - t * Q for t in range(4)]
        l = [None] * 4
        h = [None] * 4
        for t in range(4):
          x = buf[vreg(lower[t])]
          ry = jnp.flip(buf[vreg(upper[t])])
          l[t] = jnp.minimum(x, ry)
          h[t] = jnp.maximum(x, ry)
        w = [h[3 - t] for t in range(4)]
        for grp in (l, w):
          grp[0], grp[2] = jnp.minimum(grp[0], grp[2]), jnp.maximum(
              grp[0], grp[2])
          grp[1], grp[3] = jnp.minimum(grp[1], grp[3]), jnp.maximum(
              grp[1], grp[3])
          grp[0], grp[1] = jnp.minimum(grp[0], grp[1]), jnp.maximum(
              grp[0], grp[1])
          grp[2], grp[3] = jnp.minimum(grp[2], grp[3]), jnp.maximum(
              grp[2], grp[3])
        if j == 3:
          l = [jnp.sort(t) for t in l]
          w = [jnp.sort(t) for t in w]
        for t in range(4):
          buf[vreg(lower[t])] = l[t]
          buf[vreg(upper[3 - t])] = w[t]

      if j == 3:
        continue

      dists_all = []
      d = K // 8
      while d >= 1:
        dists_all.append(d)
        d //= 2
      chunks = []
      while dists_all:
        take = dists_all[-3:] if len(dists_all) >= 3 else dists_all[:]
        chunks.append(take)
        dists_all = dists_all[: len(dists_all) - len(take)]
      chunks.reverse()

      for ci, dists in enumerate(chunks):
        last = ci == len(chunks) - 1
        L = len(dists)
        G = 1 << L
        b = dists[-1].bit_length() - 1
        acc_mode = last and j == NLEV
        unroll = 2 if G >= 8 else _UNROLL

        if acc_mode:

          @plsc.parallel_loop(
              0, NV // G, unroll=unroll,
              carry=jnp.zeros((16,), jnp.float32))
          def _chunk_acc(g, acc):
            v0 = ((g >> b) << (b + L)) | (g & ((1 << b) - 1))
            xu = network(
                [buf[vreg(v0 + (t << b))] for t in range(G)], dists, True)
            xv = network(
                [buf[vreg(NV + v0 + (t << b))] for t in range(G)], dists,
                True)
            for a_u, a_v in zip(xu, xv):
              acc = acc + jnp.abs(a_u - a_v)
            return acc

          acc_final[0] = _chunk_acc
        else:

          @plsc.parallel_loop(0, 2 * NV // G, unroll=unroll)
          def _chunk(g):
            v0 = ((g >> b) << (b + L)) | (g & ((1 << b) - 1))
            xs = network(
                [buf[vreg(v0 + (t << b))] for t in range(G)], dists, last)
            for t in range(G):
              buf[vreg(v0 + (t << b))] = xs[t]

    total = jnp.sum(acc_final[0]) * (1.0 / N)
    return jnp.where(lanes == r, total, res)

  res = lax.fori_loop(0, ROWS_PER_W, row_body, jnp.zeros((16,), jnp.float32))
  res_v[...] = res
  pltpu.sync_copy(res_v, out_hbm.at[wid])


def kernel(u_values, v_values):
  mesh = plsc.VectorSubcoreMesh(core_axis_name="c", subcore_axis_name="s")
  out = pl.kernel(
      _body,
      out_type=jax.ShapeDtypeStruct((NW, 16), jnp.float32),
      mesh=mesh,
      compiler_params=pltpu.CompilerParams(needs_layout_passes=False),
      scratch_types=[
          pltpu.VMEM((2 * N,), jnp.float32),
          pltpu.VMEM((16,), jnp.float32),
      ],
  )(u_values, v_values)
  return out[:, :ROWS_PER_W].reshape(128)


# speedup vs baseline: 128.2229x; 1.0414x over previous
import jax
import jax.numpy as jnp
from jax import lax
from jax.experimental import pallas as pl
from jax.experimental.pallas import tpu as pltpu
from jax.experimental.pallas import tpu_sc as plsc

N = 8192
NV = N // 16
NLEV = 9
ROWS_PER_W = 4
NW = 32
_UNROLL = 4


def _body(u_hbm, v_hbm, out_hbm, buf, res_v):
  c = lax.axis_index("c")
  s = lax.axis_index("s")
  wid = c * 16 + s
  lanes = lax.iota(jnp.int32, 16)

  def vreg(i):
    return pl.ds(i * 16, 16)

  def row_body(r, res):
    row = wid * ROWS_PER_W + r
    pltpu.sync_copy(u_hbm.at[row], buf.at[pl.ds(0, N)])
    pltpu.sync_copy(v_hbm.at[row], buf.at[pl.ds(N, N)])

    @plsc.parallel_loop(0, NV, unroll=_UNROLL)
    def _level1(p):
      vi = 2 * p
      x = jnp.sort(buf[vreg(vi)])
      ry = jnp.flip(jnp.sort(buf[vreg(vi + 1)]))
      buf[vreg(vi)] = jnp.sort(jnp.minimum(x, ry))
      buf[vreg(vi + 1)] = jnp.sort(jnp.maximum(x, ry))

    def network(xs, dists, final_vsort):
      L = len(dists)
      xs = list(xs)
      for si in range(L):
        st = 1 << (L - 1 - si)
        for t in range(1 << L):
          if (t // st) % 2 == 0:
            a, b = xs[t], xs[t + st]
            xs[t] = jnp.minimum(a, b)
            xs[t + st] = jnp.maximum(a, b)
      if final_vsort:
        xs = [jnp.sort(x) for x in xs]
      return xs

    def a_chunk(j, La, vsort_end):
      K = 1 << (j - 1)
      H = 1 << (La - 1)
      Qa = K // H
      unroll = {2: _UNROLL, 3: 2, 4: 1}[La]

      @plsc.parallel_loop(0, NV // H, unroll=unroll)
      def _a(g):
        m = g // Qa
        i = g & (Qa - 1)
        base = m * 2 * K
        l = []
        h = []
        for t in range(H):
          x = buf[vreg(base + i + t * Qa)]
          ry = jnp.flip(buf[vreg(base + 2 * K - 1 - i - t * Qa)])
          l.append(jnp.minimum(x, ry))
          h.append(jnp.maximum(x, ry))
        w = h[::-1]
        l = network(l, [0] * (La - 1), vsort_end)
        w = network(w, [0] * (La - 1), vsort_end)
        for t in range(H):
          buf[vreg(base + i + t * Qa)] = l[t]
          buf[vreg(base + 2 * K - 1 - i - (H - 1 - t) * Qa)] = w[t]

    def chunk_pass(dists, vsort, acc_mode):
      L = len(dists)
      G = 1 << L
      b = dists[-1].bit_length() - 1
      unroll = {2: _UNROLL, 4: _UNROLL, 8: 2, 16: 1}[G]

      if acc_mode:

        @plsc.parallel_loop(
            0, NV // G, unroll=unroll, carry=jnp.zeros((16,), jnp.float32))
        def _chunk_acc(g, acc):
          v0 = ((g >> b) << (b + L)) | (g & ((1 << b) - 1))
          xu = network(
              [buf[vreg(v0 + (t << b))] for t in range(G)], dists, True)
          xv = network(
              [buf[vreg(NV + v0 + (t << b))] for t in range(G)], dists, True)
          for a_u, a_v in zip(xu, xv):
            acc = acc + jnp.abs(a_u - a_v)
          return acc

        return _chunk_acc

      @plsc.parallel_loop(0, 2 * NV // G, unroll=unroll)
      def _chunk(g):
        v0 = ((g >> b) << (b + L)) | (g & ((1 << b) - 1))
        xs = network(
            [buf[vreg(v0 + (t << b))] for t in range(G)], dists, vsort)
        for t in range(G):
          buf[vreg(v0 + (t << b))] = xs[t]

    plan = {
        5: (3, [[2, 1]]),
        6: (3, [[4, 2, 1]]),
        7: (4, [[4, 2, 1]]),
        8: (4, [[8, 4, 2, 1]]),
        9: (3, [[32, 16, 8], [4, 2, 1]]),
    }
    acc_final = [None]
    for j in range(2, NLEV + 1):
      if j <= 4:
        a_chunk(j, j, True)
        continue
      La, rem = plan[j]
      a_chunk(j, La, False)
      for ci, dists in enumerate(rem):
        last = ci == len(rem) - 1
        if last and j == NLEV:
          acc_final[0] = chunk_pass(dists, True, True)
        else:
          chunk_pass(dists, last, False)

    total = jnp.sum(acc_final[0]) * (1.0 / N)
    return jnp.where(lanes == r, total, res)

  res = lax.fori_loop(0, ROWS_PER_W, row_body, jnp.zeros((16,), jnp.float32))
  res_v[...] = res
  pltpu.sync_copy(res_v, out_hbm.at[wid])


def kernel(u_values, v_values):
  mesh = plsc.VectorSubcoreMesh(core_axis_name="c", subcore_axis_name="s")
  out = pl.kernel(
      _body,
      out_type=jax.ShapeDtypeStruct((NW, 16), jnp.float32),
      mesh=mesh,
      compiler_params=pltpu.CompilerParams(needs_layout_passes=False),
      scratch_types=[
          pltpu.VMEM((2 * N,), jnp.float32),
          pltpu.VMEM((16,), jnp.float32),
      ],
  )(u_values, v_values)
  return out[:, :ROWS_PER_W].reshape(128)
